# baseline (speedup 1.0000x reference)
import jax
import jax.numpy as jnp
from jax.experimental import pallas as pl
from jax.experimental.pallas import tpu as pltpu


def _router_block(x_ref, w1_ref, b1_ref, w2_ref, b2_ref, o_ref):
    xb = x_ref[...].astype(jnp.bfloat16)
    w1b = w1_ref[...].astype(jnp.bfloat16)
    h = jnp.dot(xb, w1b, preferred_element_type=jnp.float32)
    h = jnp.maximum(h + b1_ref[...], 0.0)
    logits = jnp.dot(h, w2_ref[...], preferred_element_type=jnp.float32)
    logits = logits + b2_ref[...]
    m = jnp.max(logits, axis=-1, keepdims=True)
    e = jnp.exp(logits - m)
    o_ref[...] = e / jnp.sum(e, axis=-1, keepdims=True)


def kernel(x, W1, b1, W2, b2):
    M, K = x.shape
    H = W1.shape[1]
    E = W2.shape[1]
    BM = 512
    grid = (M // BM,)

    b1r = b1.reshape(1, H)
    b2r = b2.reshape(1, E)

    return pl.pallas_call(
        _router_block,
        grid=grid,
        in_specs=[
            pl.BlockSpec((BM, K), lambda i: (i, 0)),
            pl.BlockSpec((K, H), lambda i: (0, 0)),
            pl.BlockSpec((1, H), lambda i: (0, 0)),
            pl.BlockSpec((H, E), lambda i: (0, 0)),
            pl.BlockSpec((1, E), lambda i: (0, 0)),
        ],
        out_specs=pl.BlockSpec((BM, E), lambda i: (i, 0)),
        out_shape=jax.ShapeDtypeStruct((M, E), jnp.float32),
        compiler_params=pltpu.CompilerParams(
            dimension_semantics=("parallel",),
        ),
    )(x, W1, b1r, W2, b2r)


# speedup vs baseline: 1.1518x; 1.1518x over previous
import jax
import jax.numpy as jnp
from jax.experimental import pallas as pl
from jax.experimental.pallas import tpu as pltpu


def _router_block(x_ref, w1_ref, b1_ref, w2_ref, b2_ref, o_ref):
    xb = x_ref[...].astype(jnp.bfloat16)
    w1b = w1_ref[...].astype(jnp.bfloat16)
    h = jnp.dot(xb, w1b, preferred_element_type=jnp.float32)
    h = jnp.maximum(h + b1_ref[...], 0.0)
    logits = jnp.dot(h, w2_ref[...], preferred_element_type=jnp.float32)
    logits = logits + b2_ref[...]
    m = jnp.max(logits, axis=-1, keepdims=True)
    e = jnp.exp(logits - m)
    o_ref[...] = e / jnp.sum(e, axis=-1, keepdims=True)


def kernel(x, W1, b1, W2, b2):
    M, K = x.shape
    H = W1.shape[1]
    E = W2.shape[1]
    BM = 1024
    grid = (M // BM,)

    b1r = b1.reshape(1, H)
    b2r = b2.reshape(1, E)

    return pl.pallas_call(
        _router_block,
        grid=grid,
        in_specs=[
            pl.BlockSpec((BM, K), lambda i: (i, 0)),
            pl.BlockSpec((K, H), lambda i: (0, 0)),
            pl.BlockSpec((1, H), lambda i: (0, 0)),
            pl.BlockSpec((H, E), lambda i: (0, 0)),
            pl.BlockSpec((1, E), lambda i: (0, 0)),
        ],
        out_specs=pl.BlockSpec((BM, E), lambda i: (i, 0)),
        out_shape=jax.ShapeDtypeStruct((M, E), jnp.float32),
        compiler_params=pltpu.CompilerParams(
            dimension_semantics=("parallel",),
        ),
    )(x, W1, b1r, W2, b2r)
